# baseline (speedup 1.0000x reference)
import functools

import jax
import jax.numpy as jnp
from jax import lax
from jax.experimental import pallas as pl
from jax.experimental.pallas import tpu as pltpu
from jax.experimental.pallas import tpu_sc as plsc


def _sc_gather_body(tidx_hbm, table_hbm, out_hbm, tidx_v, tiles_v, sem):
    wid = lax.axis_index("s") * 2 + lax.axis_index("c")

    @pl.when(wid == 0)
    def _():
        pltpu.sync_copy(tidx_hbm, tidx_v)
        n = tiles_v.shape[0] // 8
        for blk in range(n // 16):
            v = tidx_v[pl.ds(blk * 16, 16)]
            for i in range(16):
                pltpu.make_async_copy(
                    table_hbm.at[pl.ds(v[i] * 8, 8), :],
                    tiles_v.at[pl.ds((blk * 16 + i) * 8, 8), :],
                    sem,
                ).start()
        for _ in range(n):
            pltpu.make_async_copy(
                table_hbm.at[pl.ds(0, 8), :],
                tiles_v.at[pl.ds(0, 8), :],
                sem,
            ).wait()
        pltpu.sync_copy(tiles_v, out_hbm)


def _matmul_body(sub_ref, xt_ref, wlo_ref, whi_ref, b2_ref, out_ref, x2_ref):
    j = pl.program_id(0)
    n = xt_ref.shape[0] // 8
    d = xt_ref.shape[1]

    @pl.when(j == 0)
    def _build_x2():
        x2_ref[...] = jnp.zeros_like(x2_ref)
        for i in range(n):
            row = xt_ref[pl.ds(i * 8 + sub_ref[i], 1), :].astype(jnp.bfloat16)
            x2_ref[pl.ds(i, 1), pl.ds(0, d)] = row
            x2_ref[pl.ds(n + i, 1), pl.ds(d, d)] = row

    w2 = jnp.concatenate([wlo_ref[...], whi_ref[...]], axis=1)
    res = jax.lax.dot_general(
        x2_ref[...], w2,
        dimension_numbers=(((1,), (1,)), ((), ())),
        preferred_element_type=jnp.float32,
    )
    out_ref[:, 0, 0, :] = res[:n] + b2_ref[0, :][None, :]
    out_ref[:, 0, 1, :] = res[n:] + b2_ref[1, :][None, :]


def kernel(trg, enc_src, trg_mask, src_mask, emb_table, W, b):
    Bb, L = trg.shape
    V, D = emb_table.shape
    idx = trg.reshape(-1).astype(jnp.int32)
    n = idx.shape[0]
    tidx = idx // 8
    sub = idx % 8

    gather = functools.partial(
        pl.kernel,
        out_type=jax.ShapeDtypeStruct((n * 8, D), jnp.float32),
        mesh=plsc.VectorSubcoreMesh(core_axis_name="c", subcore_axis_name="s"),
        scratch_types=[
            pltpu.VMEM((n,), jnp.int32),
            pltpu.VMEM((n * 8, D), jnp.float32),
            pltpu.SemaphoreType.DMA,
        ],
        compiler_params=pltpu.CompilerParams(use_tc_tiling_on_sc=True),
    )(_sc_gather_body)
    xt = gather(tidx, emb_table)

    V2 = V // 2
    Wlo = W[:V2].astype(jnp.bfloat16)
    Whi = W[V2:].astype(jnp.bfloat16)
    b2 = b.reshape(2, V2)

    BV2 = 8192
    nv = pl.cdiv(V2, BV2)
    out = pl.pallas_call(
        _matmul_body,
        grid_spec=pltpu.PrefetchScalarGridSpec(
            num_scalar_prefetch=1,
            grid=(nv,),
            in_specs=[
                pl.BlockSpec((n * 8, D), lambda j, sub_ref: (0, 0)),
                pl.BlockSpec((BV2, D), lambda j, sub_ref: (j, 0)),
                pl.BlockSpec((BV2, D), lambda j, sub_ref: (j, 0)),
                pl.BlockSpec((2, BV2), lambda j, sub_ref: (0, j)),
            ],
            out_specs=pl.BlockSpec((n, 1, 2, BV2),
                                   lambda j, sub_ref: (0, 0, 0, j)),
            scratch_shapes=[
                pltpu.VMEM((2 * n, 2 * D), jnp.bfloat16),
            ],
        ),
        out_shape=jax.ShapeDtypeStruct((n, 1, 2, V2), jnp.float32),
        compiler_params=pltpu.CompilerParams(
            dimension_semantics=("arbitrary",),
        ),
    )(sub, xt, Wlo, Whi, b2)
    return out.reshape(Bb, L, V)


# speedup vs baseline: 1.5086x; 1.5086x over previous
import functools

import jax
import jax.numpy as jnp
from jax import lax
from jax.experimental import pallas as pl
from jax.experimental.pallas import tpu as pltpu
from jax.experimental.pallas import tpu_sc as plsc


def _sc_gather_body(tidx_hbm, table_hbm, out_hbm, tidx_row, tile_v, sem):
    wid = lax.axis_index("s") * 2 + lax.axis_index("c")
    pltpu.sync_copy(tidx_hbm.at[pl.ds(wid, 1), :], tidx_row)
    t = jnp.reshape(tidx_row[...], (tidx_row.shape[1],))[0]
    pltpu.make_async_copy(
        table_hbm.at[pl.ds(t * 8, 8), :], tile_v, sem).start()
    pltpu.make_async_copy(
        table_hbm.at[pl.ds(0, 8), :], tile_v, sem).wait()
    pltpu.sync_copy(tile_v, out_hbm.at[pl.ds(wid * 8, 8), :])


def _matmul_body(sub_ref, xt_ref, wlo_ref, whi_ref, b_ref, out_ref, x2_ref):
    j = pl.program_id(0)
    n = xt_ref.shape[0] // 8
    d = xt_ref.shape[1]
    bv = wlo_ref.shape[0]

    @pl.when(j == 0)
    def _build_x2():
        x2_ref[...] = jnp.zeros_like(x2_ref)
        for i in range(n):
            row = xt_ref[pl.ds(i * 8 + sub_ref[i], 1), :].astype(jnp.bfloat16)
            x2_ref[pl.ds(i, 1), pl.ds(0, d)] = row
            x2_ref[pl.ds(n + i, 1), pl.ds(d, d)] = row

    w2 = jnp.concatenate([wlo_ref[...], whi_ref[...]], axis=1)
    res = jax.lax.dot_general(
        x2_ref[...], w2,
        dimension_numbers=(((1,), (1,)), ((), ())),
        preferred_element_type=jnp.float32,
    )
    out_ref[:, 0, pl.ds(0, bv)] = res[:n] + b_ref[pl.ds(0, bv)][None, :]
    out_ref[:, 0, pl.ds(bv, bv)] = res[n:] + b_ref[pl.ds(bv, bv)][None, :]


def kernel(trg, enc_src, trg_mask, src_mask, emb_table, W, b):
    Bb, L = trg.shape
    V, D = emb_table.shape
    idx = trg.reshape(-1).astype(jnp.int32)
    n = idx.shape[0]
    tidx_rep = jnp.broadcast_to((idx // 8)[:, None], (n, 16))
    sub = idx % 8

    gather = functools.partial(
        pl.kernel,
        out_type=jax.ShapeDtypeStruct((n * 8, D), jnp.float32),
        mesh=plsc.VectorSubcoreMesh(core_axis_name="c", subcore_axis_name="s"),
        scratch_types=[
            pltpu.VMEM((1, 16), jnp.int32),
            pltpu.VMEM((8, D), jnp.float32),
            pltpu.SemaphoreType.DMA,
        ],
        compiler_params=pltpu.CompilerParams(use_tc_tiling_on_sc=True),
    )(_sc_gather_body)
    xt = gather(tidx_rep, emb_table)

    Wbf = W.astype(jnp.bfloat16)

    BV = 8192
    nv = pl.cdiv(V, 2 * BV)
    nbw = pl.cdiv(V, BV)
    out = pl.pallas_call(
        _matmul_body,
        grid_spec=pltpu.PrefetchScalarGridSpec(
            num_scalar_prefetch=1,
            grid=(nv,),
            in_specs=[
                pl.BlockSpec((n * 8, D), lambda j, sub_ref: (0, 0)),
                pl.BlockSpec((BV, D), lambda j, sub_ref: (2 * j, 0)),
                pl.BlockSpec(
                    (BV, D),
                    lambda j, sub_ref: (jnp.minimum(2 * j + 1, nbw - 1), 0)),
                pl.BlockSpec((2 * BV,), lambda j, sub_ref: (j,)),
            ],
            out_specs=pl.BlockSpec((n, 1, 2 * BV),
                                   lambda j, sub_ref: (0, 0, j)),
            scratch_shapes=[
                pltpu.VMEM((2 * n, 2 * D), jnp.bfloat16),
            ],
        ),
        out_shape=jax.ShapeDtypeStruct((n, 1, V), jnp.float32),
        compiler_params=pltpu.CompilerParams(
            dimension_semantics=("arbitrary",),
        ),
    )(sub, xt, Wbf, Wbf, b)
    return out.reshape(Bb, L, V)


# speedup vs baseline: 9.5797x; 6.3500x over previous
import jax
import jax.numpy as jnp
from jax.experimental import pallas as pl
from jax.experimental.pallas import tpu as pltpu


def _body(idx_ref, et_hbm, wlo_ref, whi_ref, b_ref, out_ref,
          xg_ref, x2t_ref, sem):
    j = pl.program_id(0)
    n = out_ref.shape[0]
    d = wlo_ref.shape[0]
    bv = wlo_ref.shape[1]

    @pl.when(j == 0)
    def _gather_and_pack():
        for i in range(n):
            base = (idx_ref[i] // 128) * 128
            pltpu.make_async_copy(
                et_hbm.at[:, pl.ds(base, 128)],
                xg_ref.at[:, pl.ds(i * 128, 128)],
                sem,
            ).start()
        for _ in range(n):
            pltpu.make_async_copy(
                et_hbm.at[:, pl.ds(0, 128)],
                xg_ref.at[:, pl.ds(0, 128)],
                sem,
            ).wait()
        x2t_ref[...] = jnp.zeros_like(x2t_ref)
        lanes = jax.lax.broadcasted_iota(jnp.int32, (d, 128), 1)
        for i in range(n):
            sub = idx_ref[i] % 128
            win = xg_ref[:, pl.ds(i * 128, 128)]
            col = jnp.sum(jnp.where(lanes == sub, win, 0.0),
                          axis=1, keepdims=True).astype(jnp.bfloat16)
            x2t_ref[pl.ds(0, d), pl.ds(i, 1)] = col
            x2t_ref[pl.ds(d, d), pl.ds(n + i, 1)] = col

    w2 = jnp.concatenate(
        [wlo_ref[...].astype(jnp.bfloat16), whi_ref[...].astype(jnp.bfloat16)],
        axis=0)
    res = jax.lax.dot_general(
        x2t_ref[...], w2,
        dimension_numbers=(((0,), (0,)), ((), ())),
        preferred_element_type=jnp.float32,
    )
    out_ref[:, 0, pl.ds(0, bv)] = res[:n] + b_ref[pl.ds(0, bv)][None, :]
    out_ref[:, 0, pl.ds(bv, bv)] = res[n:] + b_ref[pl.ds(bv, bv)][None, :]


def kernel(trg, enc_src, trg_mask, src_mask, emb_table, W, b):
    Bb, L = trg.shape
    V, D = emb_table.shape
    idx = trg.reshape(-1).astype(jnp.int32)
    n = idx.shape[0]

    Et = emb_table.T
    Wt = W.T

    BV = 8192
    nv = pl.cdiv(V, 2 * BV)
    nbw = pl.cdiv(V, BV)
    out = pl.pallas_call(
        _body,
        grid_spec=pltpu.PrefetchScalarGridSpec(
            num_scalar_prefetch=1,
            grid=(nv,),
            in_specs=[
                pl.BlockSpec(memory_space=pltpu.MemorySpace.HBM),
                pl.BlockSpec((D, BV), lambda j, idx_ref: (0, 2 * j)),
                pl.BlockSpec(
                    (D, BV),
                    lambda j, idx_ref: (0, jnp.minimum(2 * j + 1, nbw - 1))),
                pl.BlockSpec((2 * BV,), lambda j, idx_ref: (j,)),
            ],
            out_specs=pl.BlockSpec((n, 1, 2 * BV),
                                   lambda j, idx_ref: (0, 0, j)),
            scratch_shapes=[
                pltpu.VMEM((D, n * 128), jnp.float32),
                pltpu.VMEM((2 * D, 2 * n), jnp.bfloat16),
                pltpu.SemaphoreType.DMA,
            ],
        ),
        out_shape=jax.ShapeDtypeStruct((n, 1, V), jnp.float32),
        compiler_params=pltpu.CompilerParams(
            dimension_semantics=("arbitrary",),
        ),
    )(idx, Et, Wt, Wt, b)
    return out.reshape(Bb, L, V)


# speedup vs baseline: 9.9805x; 1.0418x over previous
import jax
import jax.numpy as jnp
from jax.experimental import pallas as pl
from jax.experimental.pallas import tpu as pltpu


def _body(idx_ref, et_hbm, wlo_ref, whi_ref, b_ref, out_ref,
          xg_ref, x2t_ref, sem):
    j = pl.program_id(0)
    n = out_ref.shape[0]
    d = wlo_ref.shape[0]
    bv = wlo_ref.shape[1]

    @pl.when(j == 0)
    def _gather_and_pack():
        for i in range(n):
            base = (idx_ref[i] // 128) * 128
            pltpu.make_async_copy(
                et_hbm.at[:, pl.ds(base, 128)],
                xg_ref.at[:, pl.ds(i * 128, 128)],
                sem,
            ).start()
        for _ in range(n):
            pltpu.make_async_copy(
                et_hbm.at[:, pl.ds(0, 128)],
                xg_ref.at[:, pl.ds(0, 128)],
                sem,
            ).wait()
        x2t_ref[...] = jnp.zeros_like(x2t_ref)
        lanes = jax.lax.broadcasted_iota(jnp.int32, (d, 128), 1)
        for i in range(n):
            sub = idx_ref[i] % 128
            win = xg_ref[:, pl.ds(i * 128, 128)]
            col = jnp.sum(jnp.where(lanes == sub, win, 0.0),
                          axis=1, keepdims=True).astype(jnp.bfloat16)
            x2t_ref[pl.ds(0, d), pl.ds(i, 1)] = col
            x2t_ref[pl.ds(d, d), pl.ds(n + i, 1)] = col

    w2 = jnp.concatenate(
        [wlo_ref[...].astype(jnp.bfloat16), whi_ref[...].astype(jnp.bfloat16)],
        axis=0)
    res = jax.lax.dot_general(
        x2t_ref[...], w2,
        dimension_numbers=(((0,), (0,)), ((), ())),
        preferred_element_type=jnp.float32,
    )
    out_ref[:, 0, pl.ds(0, bv)] = res[:n] + b_ref[pl.ds(0, bv)][None, :]
    out_ref[:, 0, pl.ds(bv, bv)] = res[n:] + b_ref[pl.ds(bv, bv)][None, :]


def kernel(trg, enc_src, trg_mask, src_mask, emb_table, W, b):
    Bb, L = trg.shape
    V, D = emb_table.shape
    idx = trg.reshape(-1).astype(jnp.int32)
    n = idx.shape[0]

    Et = emb_table.T
    Wt = W.T

    BV = 16384
    nv = pl.cdiv(V, 2 * BV)
    nbw = pl.cdiv(V, BV)
    out = pl.pallas_call(
        _body,
        grid_spec=pltpu.PrefetchScalarGridSpec(
            num_scalar_prefetch=1,
            grid=(nv,),
            in_specs=[
                pl.BlockSpec(memory_space=pltpu.MemorySpace.HBM),
                pl.BlockSpec((D, BV), lambda j, idx_ref: (0, 2 * j)),
                pl.BlockSpec(
                    (D, BV),
                    lambda j, idx_ref: (0, jnp.minimum(2 * j + 1, nbw - 1))),
                pl.BlockSpec((2 * BV,), lambda j, idx_ref: (j,)),
            ],
            out_specs=pl.BlockSpec((n, 1, 2 * BV),
                                   lambda j, idx_ref: (0, 0, j)),
            scratch_shapes=[
                pltpu.VMEM((D, n * 128), jnp.float32),
                pltpu.VMEM((2 * D, 2 * n), jnp.bfloat16),
                pltpu.SemaphoreType.DMA,
            ],
        ),
        out_shape=jax.ShapeDtypeStruct((n, 1, V), jnp.float32),
        compiler_params=pltpu.CompilerParams(
            dimension_semantics=("arbitrary",),
        ),
    )(idx, Et, Wt, Wt, b)
    return out.reshape(Bb, L, V)
